# baseline (device time: 198701 ns/iter reference)
import jax
import jax.numpy as jnp
from jax import lax
from jax.experimental import pallas as pl
from jax.experimental.pallas import tpu as pltpu

_K = 8
_D_Y = (0, 1, 2)
_D_X = (3, 5)
_D_Z = (4, 6, 7)
_KY = _K + len(_D_Y)


def kernel(x):
    m, n = x.shape
    quarter = m // 4
    ch = quarter // _K

    def body(
        x_hbm,
        out_ref,
        y_q,
        recv,
        xin,
        o_slots,
        y_send, y_recv,
        x_send, x_recv,
        z_send, z_recv,
        xin_sem, o_out,
    ):
        my_x = lax.axis_index("x")
        my_y = lax.axis_index("y")
        my_z = lax.axis_index("z")
        zp = lax.rem(my_z, 2)
        partner = (my_x, 1 - my_y, my_z)
        xnbr = (1 - my_x, my_y, my_z)
        znbr = (my_x, my_y, my_z + 1 - 2 * zp)

        q_own = (2 * my_x + zp) * quarter
        q_x = (2 * (1 - my_x) + zp) * quarter
        q_z = (2 * my_x + (1 - zp)) * quarter
        q_d = (2 * (1 - my_x) + (1 - zp)) * quarter

        def y_row(k):
            if k < _K:
                return q_own + k * ch
            return q_d + _D_Y[k - _K] * ch

        barrier_sem = pltpu.get_barrier_semaphore()
        for nbr in (partner, xnbr, znbr):
            pl.semaphore_signal(
                barrier_sem, inc=1, device_id=nbr,
                device_id_type=pl.DeviceIdType.MESH,
            )

        cp = pltpu.make_async_copy(
            x_hbm.at[pl.ds(y_row(0), ch), :], xin.at[0], xin_sem.at[0]
        )
        cp.start()
        pl.semaphore_wait(barrier_sem, 3)

        prev = cp
        y_rdmas = []
        for k in range(_KY):
            slot = k % 2
            prev.wait()
            if k + 1 < _KY:
                nxt = pltpu.make_async_copy(
                    x_hbm.at[pl.ds(y_row(k + 1), ch), :],
                    xin.at[(k + 1) % 2],
                    xin_sem.at[(k + 1) % 2],
                )
                nxt.start()
                prev = nxt
            y_q[k * ch:(k + 1) * ch, :] = xin[slot].astype(jnp.bfloat16)
            r = pltpu.make_async_remote_copy(
                src_ref=y_q.at[pl.ds(k * ch, ch), :],
                dst_ref=recv.at[pl.ds(y_row(k), ch), :],
                send_sem=y_send.at[k],
                recv_sem=y_recv.at[k],
                device_id=partner,
                device_id_type=pl.DeviceIdType.MESH,
            )
            r.start()
            y_rdmas.append(r)

        events = []
        d_seen = 0
        d_order = sorted(
            [(j, "dx") for j in _D_X] + [(j, "dz") for j in _D_Z]
        )
        for k in range(_K):
            events.append(("y", k))
            if k >= 1:
                events.append(("x", k - 1))
                events.append(("z", k - 1))
            if k >= 4 and d_seen < len(d_order):
                events.append((d_order[d_seen][1], d_order[d_seen][0]))
                d_seen += 1
        events += [("x", _K - 1), ("z", _K - 1)]
        events += [("y", _K), ("y", _K + 1)]
        while d_seen < len(d_order):
            events.append((d_order[d_seen][1], d_order[d_seen][0]))
            d_seen += 1
        events.append(("y", _K + 2))
        assert len(events) == 4 * _K

        def ev_row(ev):
            kind, k = ev
            if kind == "y":
                return y_row(k)
            return {"x": q_x, "z": q_z, "dx": q_d, "dz": q_d}[kind] + k * ch

        non_y = [ev for ev in events if ev[0] != "y"]
        xin_cps = {}

        def stage(p):
            if p < len(non_y):
                xcp = pltpu.make_async_copy(
                    x_hbm.at[pl.ds(ev_row(non_y[p]), ch), :],
                    xin.at[p % 2],
                    xin_sem.at[p % 2],
                )
                xcp.start()
                xin_cps[p] = xcp

        stage(0)
        stage(1)

        o_cps = []
        emitted = [0]
        x_relays = []
        z_relays = []

        def emit(row_start, a_vec):
            slot = emitted[0] % 2
            if len(o_cps) >= 2:
                o_cps[-2].wait()
            o_slots[slot, :, :] = a_vec + recv[pl.ds(row_start, ch), :]
            ocp = pltpu.make_async_copy(
                o_slots.at[slot],
                out_ref.at[pl.ds(row_start, ch), :],
                o_out.at[slot],
            )
            ocp.start()
            o_cps.append(ocp)
            emitted[0] += 1

        def relay(row_start, target, sems_send, sems_recv, idx, bucket):
            r = pltpu.make_async_remote_copy(
                src_ref=recv.at[pl.ds(row_start, ch), :],
                dst_ref=recv.at[pl.ds(row_start, ch), :],
                send_sem=sems_send.at[idx],
                recv_sem=sems_recv.at[idx],
                device_id=target,
                device_id_type=pl.DeviceIdType.MESH,
            )
            r.start()
            bucket.append(r)

        def wait_recv_only(row_start, sems_recv, idx):
            rx = pltpu.make_async_remote_copy(
                src_ref=recv.at[pl.ds(row_start, ch), :],
                dst_ref=recv.at[pl.ds(row_start, ch), :],
                send_sem=y_send.at[0],
                recv_sem=sems_recv.at[idx],
                device_id=partner,
                device_id_type=pl.DeviceIdType.MESH,
            )
            rx.wait_recv()

        p = 0
        for ev in events:
            kind, k = ev
            row = ev_row(ev)
            if kind == "y":
                y_rdmas[k].wait_recv()
                if k < _K:
                    relay(row, xnbr, x_send, x_recv, k, x_relays)
                    relay(row, znbr, z_send, z_recv, k, z_relays)
                emit(row, y_q[pl.ds(k * ch, ch), :])
            else:
                if kind == "x":
                    wait_recv_only(row, x_recv, k)
                    if k in _D_Z:
                        relay(row, znbr, z_send, z_recv,
                              _K + _D_Z.index(k), z_relays)
                elif kind == "z":
                    wait_recv_only(row, z_recv, k)
                    if k in _D_X:
                        relay(row, xnbr, x_send, x_recv,
                              _K + _D_X.index(k), x_relays)
                elif kind == "dx":
                    wait_recv_only(row, x_recv, _K + _D_X.index(k))
                else:
                    wait_recv_only(row, z_recv, _K + _D_Z.index(k))
                xin_cps[p].wait()
                emit(row, xin[p % 2].astype(jnp.bfloat16))
                stage(p + 2)
                p += 1

        o_cps[-2].wait()
        o_cps[-1].wait()
        for r in y_rdmas:
            r.wait_send()
        for r in x_relays:
            r.wait_send()
        for r in z_relays:
            r.wait_send()

    return pl.pallas_call(
        body,
        out_shape=jax.ShapeDtypeStruct((m, n), jnp.bfloat16),
        in_specs=[pl.BlockSpec(memory_space=pl.ANY)],
        out_specs=pl.BlockSpec(memory_space=pl.ANY),
        scratch_shapes=[
            pltpu.VMEM((_KY * ch, n), jnp.bfloat16),
            pltpu.VMEM((m, n), jnp.bfloat16),
            pltpu.VMEM((2, ch, n), jnp.float32),
            pltpu.VMEM((2, ch, n), jnp.bfloat16),
            pltpu.SemaphoreType.DMA((_KY,)),
            pltpu.SemaphoreType.DMA((_KY,)),
            pltpu.SemaphoreType.DMA((_K + len(_D_X),)),
            pltpu.SemaphoreType.DMA((_K + len(_D_X),)),
            pltpu.SemaphoreType.DMA((_K + len(_D_Z),)),
            pltpu.SemaphoreType.DMA((_K + len(_D_Z),)),
            pltpu.SemaphoreType.DMA((2,)),
            pltpu.SemaphoreType.DMA((2,)),
        ],
        compiler_params=pltpu.CompilerParams(
            collective_id=0,
            vmem_limit_bytes=56 * 1024 * 1024,
        ),
    )(x)
